# baseline (speedup 1.0000x reference)
import functools

import jax
import jax.numpy as jnp
from jax import lax
from jax.experimental import pallas as pl
from jax.experimental.pallas import tpu as pltpu
from jax.experimental.pallas import tpu_sc as plsc

_EMB = 128
_N = 100000
_T1_ROWS = 4 * 12 * 12
_T2_ROWS = 10 * 6 * 6 * 2 * 2

_NC, _NS = 2, 16
_NW = _NC * _NS
_B = 128
_NF = 9
_FULL_CHUNKS = _N // _B
_REM = _N - _FULL_CHUNKS * _B


def _build_tables_body(w1, w2, w3, w4, w5, w6, w7, w8, t1, t2):
    a = (w1[...][:, None, :] + w2[...][None, :, :]).reshape(48, _EMB)
    t1[...] = (a[:, None, :] + w3[...][None, :, :]).reshape(_T1_ROWS, _EMB)
    b = (w4[...][:, None, :] + w5[...][None, :, :]).reshape(60, _EMB)
    b = (b[:, None, :] + w6[...][None, :, :]).reshape(360, _EMB)
    b = (b[:, None, :] + w7[...][None, :, :]).reshape(720, _EMB)
    t2[...] = (b[:, None, :] + w8[...][None, :, :]).reshape(_T2_ROWS, _EMB)


def _build_tables(w1, w2, w3, w4, w5, w6, w7, w8):
    return pl.pallas_call(
        _build_tables_body,
        out_shape=[
            jax.ShapeDtypeStruct((_T1_ROWS, _EMB), jnp.float32),
            jax.ShapeDtypeStruct((_T2_ROWS, _EMB), jnp.float32),
        ],
    )(w1, w2, w3, w4, w5, w6, w7, w8)


def _fuse_indices(xv, c0v, c1v, c2v, nb):
    nn = nb * 16
    for t in range(nb):
        sl = pl.ds(t * 16, 16)
        xi = [xv[pl.ds(i * nn + t * 16, 16)] for i in range(_NF)]
        c0 = xi[0]
        c1 = (xi[1] * 12 + xi[2]) * 12 + xi[3]
        c2 = (((xi[4] * 6 + xi[5]) * 6 + xi[6]) * 2 + xi[7]) * 2 + xi[8]
        c0v[sl] = c0
        c1v[sl] = c1
        c2v[sl] = c2


def _chunk_body(node_base, nb, xt, w0, t1, t2, out,
                xv, c0v, c1v, c2v, r0, r1, r2, sem):
    nn = nb * 16
    noff = pl.multiple_of(node_base, 8)
    for i in range(_NF):
        pltpu.sync_copy(xt.at[pl.ds(i * _N + noff, nn)], xv.at[pl.ds(i * nn, nn)])
    _fuse_indices(xv, c0v, c1v, c2v, nb)
    cp0 = pltpu.async_copy(w0.at[c0v], r0, sem)
    cp1 = pltpu.async_copy(t1.at[c1v], r1, sem)
    cp2 = pltpu.async_copy(t2.at[c2v], r2, sem)
    cp0.wait()
    cp1.wait()
    cp2.wait()

    def acc_row(r, _):
        for l in range(_EMB // 16):
            sl = pl.ds(l * 16, 16)
            r0[r, sl] = r0[r, sl] + r1[r, sl] + r2[r, sl]
        return 0

    lax.fori_loop(0, nn, acc_row, 0)
    pltpu.sync_copy(r0, out.at[pl.ds(node_base, nn)])


def _sc_lookup(xt, w0, t1, t2):
    mesh = plsc.VectorSubcoreMesh(
        core_axis_name="c", subcore_axis_name="s",
        num_cores=_NC, num_subcores=_NS)

    @functools.partial(
        pl.kernel,
        out_type=jax.ShapeDtypeStruct((_N, _EMB), jnp.float32),
        mesh=mesh,
        scratch_types=dict(
            xv=pltpu.VMEM((_NF * _B,), jnp.int32),
            c0v=pltpu.VMEM((_B,), jnp.int32),
            c1v=pltpu.VMEM((_B,), jnp.int32),
            c2v=pltpu.VMEM((_B,), jnp.int32),
            r0=pltpu.VMEM((_B, _EMB), jnp.float32),
            r1=pltpu.VMEM((_B, _EMB), jnp.float32),
            r2=pltpu.VMEM((_B, _EMB), jnp.float32),
            xvr=pltpu.VMEM((_NF * _REM,), jnp.int32),
            c0r=pltpu.VMEM((_REM,), jnp.int32),
            c1r=pltpu.VMEM((_REM,), jnp.int32),
            c2r=pltpu.VMEM((_REM,), jnp.int32),
            q0=pltpu.VMEM((_REM, _EMB), jnp.float32),
            q1=pltpu.VMEM((_REM, _EMB), jnp.float32),
            q2=pltpu.VMEM((_REM, _EMB), jnp.float32),
            sem=pltpu.SemaphoreType.DMA,
        ),
    )
    def k(xt_hbm, w0_hbm, t1_hbm, t2_hbm, out_hbm,
          xv, c0v, c1v, c2v, r0, r1, r2,
          xvr, c0r, c1r, c2r, q0, q1, q2, sem):
        wid = lax.axis_index("s") * _NC + lax.axis_index("c")
        nj = (_FULL_CHUNKS - wid + _NW - 1) // _NW

        def chunk(j, _):
            c = wid + j * _NW
            _chunk_body(c * _B, _B // 16, xt_hbm, w0_hbm, t1_hbm, t2_hbm,
                        out_hbm, xv, c0v, c1v, c2v, r0, r1, r2, sem)
            return 0

        lax.fori_loop(0, nj, chunk, 0)

        @pl.when(wid == _NW - 1)
        def _():
            _chunk_body(_FULL_CHUNKS * _B, _REM // 16, xt_hbm, w0_hbm,
                        t1_hbm, t2_hbm, out_hbm,
                        xvr, c0r, c1r, c2r, q0, q1, q2, sem)

    return k(xt, w0, t1, t2)


def kernel(x, W0, W1, W2, W3, W4, W5, W6, W7, W8):
    xt = x.astype(jnp.int32).T.reshape(-1)
    t1, t2 = _build_tables(W1, W2, W3, W4, W5, W6, W7, W8)
    return _sc_lookup(xt, W0, t1, t2)
